# baseline (speedup 1.0000x reference)
import functools

import jax
import jax.numpy as jnp
from jax import lax
from jax.experimental import pallas as pl
from jax.experimental.pallas import tpu as pltpu
from jax.experimental.pallas import tpu_sc as plsc

_NUM_CORES = 2
_NUM_SUBCORES = 16
_NUM_WORKERS = _NUM_CORES * _NUM_SUBCORES
_LANES = 16


def _logits_body(x_ref, w_ref, out_ref):
    out_ref[...] = jax.lax.dot_general(
        x_ref[...], w_ref[...], (((1,), (1,)), ((), ())),
        preferred_element_type=jnp.float32,
    )


@functools.partial(jax.jit, static_argnames=("tt",))
def _logits_tc(x, w, tt):
    tokens, d = x.shape
    e = w.shape[0]
    return pl.pallas_call(
        _logits_body,
        grid=(tokens // tt,),
        in_specs=[
            pl.BlockSpec((tt, d), lambda i: (i, 0)),
            pl.BlockSpec((e, d), lambda i: (0, 0)),
        ],
        out_specs=pl.BlockSpec((tt, e), lambda i: (i, 0)),
        out_shape=jax.ShapeDtypeStruct((tokens, e), jnp.float32),
        compiler_params=pltpu.CompilerParams(
            dimension_semantics=("arbitrary",),
        ),
    )(x, w)


def _make_sc_router(tokens, e):
    tpw = tokens // _NUM_WORKERS
    mesh = plsc.VectorSubcoreMesh(core_axis_name="c", subcore_axis_name="s")

    @functools.partial(
        pl.kernel,
        mesh=mesh,
        out_type=[
            jax.ShapeDtypeStruct((tokens, e), jnp.float32),
            jax.ShapeDtypeStruct((tokens, e), jnp.float32),
        ],
        scratch_types=[
            pltpu.VMEM((tpw, e), jnp.float32),
            pltpu.VMEM((tpw, e), jnp.float32),
            pltpu.VMEM((tpw, e), jnp.float32),
            pltpu.SemaphoreType.DMA,
        ],
        compiler_params=pltpu.CompilerParams(needs_layout_passes=False),
    )
    def _sc_route(logits_hbm, probs_hbm, mask_hbm, lbuf, pbuf, mbuf, sem):
        wid = lax.axis_index("s") * _NUM_CORES + lax.axis_index("c")
        base = wid * tpw
        pltpu.async_copy(logits_hbm.at[pl.ds(base, tpw)], lbuf, sem).wait()

        iota = lax.broadcasted_iota(jnp.int32, (_LANES,), 0)
        neg_inf = jnp.full((_LANES,), -jnp.inf, jnp.float32)
        zero = jnp.zeros((_LANES,), jnp.float32)
        one = jnp.ones((_LANES,), jnp.float32)
        big = jnp.full((_LANES,), e, jnp.int32)

        @pl.loop(0, tpw // _LANES)
        def _(g):
            rows = g * _LANES + iota
            cols = [jnp.full((_LANES,), ee, jnp.int32) for ee in range(e)]
            v = [plsc.load_gather(lbuf, [rows, cols[ee]]) for ee in range(e)]
            m1 = v[0]
            for ee in range(1, e):
                m1 = jnp.maximum(m1, v[ee])
            idx1 = big
            for ee in range(e):
                idx1 = jnp.minimum(
                    idx1, jnp.where(v[ee] == m1, cols[ee], big)
                )
            sel1 = [idx1 == ee for ee in range(e)]
            v2 = [jnp.where(sel1[ee], neg_inf, v[ee]) for ee in range(e)]
            m2 = v2[0]
            for ee in range(1, e):
                m2 = jnp.maximum(m2, v2[ee])
            idx2 = big
            for ee in range(e):
                idx2 = jnp.minimum(
                    idx2, jnp.where(v2[ee] == m2, cols[ee], big)
                )
            tv = jnp.exp(m2 - m1)
            denom = 1.0 + tv
            p1 = 1.0 / denom
            p2 = tv / denom
            for ee in range(e):
                s1 = sel1[ee]
                s2 = idx2 == ee
                pe = jnp.where(s1, p1, jnp.where(s2, p2, zero))
                me = jnp.where(s1 | s2, one, zero)
                plsc.store_scatter(pbuf, [rows, cols[ee]], pe)
                plsc.store_scatter(mbuf, [rows, cols[ee]], me)

        pltpu.async_copy(pbuf, probs_hbm.at[pl.ds(base, tpw)], sem).wait()
        pltpu.async_copy(mbuf, mask_hbm.at[pl.ds(base, tpw)], sem).wait()

    return _sc_route


@jax.jit
def _route_hybrid(x, w):
    tokens, _ = x.shape
    e = w.shape[0]
    logits = _logits_tc(x, w, tt=1024)
    probs, mask = _make_sc_router(tokens, e)(logits)
    return probs, mask.astype(jnp.bool_)


def kernel(hidden_states, router_weight):
    s, b, d = hidden_states.shape
    x = hidden_states.reshape(s * b, d).astype(jnp.float32)
    return _route_hybrid(x, router_weight.astype(jnp.float32))


# speedup vs baseline: 1.0138x; 1.0138x over previous
import functools

import jax
import jax.numpy as jnp
from jax import lax
from jax.experimental import pallas as pl
from jax.experimental.pallas import tpu as pltpu
from jax.experimental.pallas import tpu_sc as plsc

_NUM_CORES = 2
_NUM_SUBCORES = 16
_NUM_WORKERS = _NUM_CORES * _NUM_SUBCORES
_LANES = 16


def _top2(logits, e):
    tt = logits.shape[0]
    iota = jax.lax.broadcasted_iota(jnp.int32, (tt, e), 1)
    m1 = jnp.max(logits, axis=1, keepdims=True)
    idx1 = jnp.min(jnp.where(logits == m1, iota, e), axis=1, keepdims=True)
    masked = jnp.where(iota == idx1, -jnp.inf, logits)
    m2 = jnp.max(masked, axis=1, keepdims=True)
    idx2 = jnp.min(jnp.where(masked == m2, iota, e), axis=1, keepdims=True)
    return iota, m1, idx1, m2, idx2


def _logits_body(x_ref, w_ref, out_ref, map_ref):
    logits = jax.lax.dot_general(
        x_ref[...], w_ref[...], (((1,), (1,)), ((), ())),
        preferred_element_type=jnp.float32,
    )
    out_ref[...] = logits
    iota, _, idx1, _, idx2 = _top2(logits, logits.shape[1])
    map_ref[...] = (iota == idx1) | (iota == idx2)


@functools.partial(jax.jit, static_argnames=("tt",))
def _logits_tc(x, w, tt):
    tokens, d = x.shape
    e = w.shape[0]
    return pl.pallas_call(
        _logits_body,
        grid=(tokens // tt,),
        in_specs=[
            pl.BlockSpec((tt, d), lambda i: (i, 0)),
            pl.BlockSpec((e, d), lambda i: (0, 0)),
        ],
        out_specs=[
            pl.BlockSpec((tt, e), lambda i: (i, 0)),
            pl.BlockSpec((tt, e), lambda i: (i, 0)),
        ],
        out_shape=[
            jax.ShapeDtypeStruct((tokens, e), jnp.float32),
            jax.ShapeDtypeStruct((tokens, e), jnp.bool_),
        ],
        compiler_params=pltpu.CompilerParams(
            dimension_semantics=("arbitrary",),
        ),
    )(x, w)


def _make_sc_router(tokens, e):
    tpw = tokens // _NUM_WORKERS
    mesh = plsc.VectorSubcoreMesh(core_axis_name="c", subcore_axis_name="s")

    @functools.partial(
        pl.kernel,
        mesh=mesh,
        out_type=jax.ShapeDtypeStruct((tokens, e), jnp.float32),
        scratch_types=[
            pltpu.VMEM((tpw, e), jnp.float32),
            pltpu.VMEM((tpw, e), jnp.float32),
            pltpu.SemaphoreType.DMA,
        ],
        compiler_params=pltpu.CompilerParams(needs_layout_passes=False),
    )
    def _sc_route(logits_hbm, probs_hbm, lbuf, pbuf, sem):
        wid = lax.axis_index("s") * _NUM_CORES + lax.axis_index("c")
        base = wid * tpw
        pltpu.async_copy(logits_hbm.at[pl.ds(base, tpw)], lbuf, sem).wait()

        iota = lax.broadcasted_iota(jnp.int32, (_LANES,), 0)
        neg_inf = jnp.full((_LANES,), -jnp.inf, jnp.float32)
        zero = jnp.zeros((_LANES,), jnp.float32)
        big = jnp.full((_LANES,), e, jnp.int32)

        @pl.loop(0, tpw // _LANES)
        def _(g):
            rows = g * _LANES + iota
            cols = [jnp.full((_LANES,), ee, jnp.int32) for ee in range(e)]
            v = [plsc.load_gather(lbuf, [rows, cols[ee]]) for ee in range(e)]
            m1 = v[0]
            for ee in range(1, e):
                m1 = jnp.maximum(m1, v[ee])
            idx1 = big
            for ee in range(e):
                idx1 = jnp.minimum(
                    idx1, jnp.where(v[ee] == m1, cols[ee], big)
                )
            sel1 = [idx1 == ee for ee in range(e)]
            v2 = [jnp.where(sel1[ee], neg_inf, v[ee]) for ee in range(e)]
            m2 = v2[0]
            for ee in range(1, e):
                m2 = jnp.maximum(m2, v2[ee])
            idx2 = big
            for ee in range(e):
                idx2 = jnp.minimum(
                    idx2, jnp.where(v2[ee] == m2, cols[ee], big)
                )
            tv = jnp.exp(m2 - m1)
            denom = 1.0 + tv
            p1 = 1.0 / denom
            p2 = tv / denom
            for ee in range(e):
                pe = jnp.where(sel1[ee], p1, jnp.where(idx2 == ee, p2, zero))
                plsc.store_scatter(pbuf, [rows, cols[ee]], pe)

        pltpu.async_copy(pbuf, probs_hbm.at[pl.ds(base, tpw)], sem).wait()

    return _sc_route


@jax.jit
def _route_hybrid(x, w):
    tokens, _ = x.shape
    e = w.shape[0]
    logits, routing_map = _logits_tc(x, w, tt=1024)
    probs = _make_sc_router(tokens, e)(logits)
    return probs, routing_map


def kernel(hidden_states, router_weight):
    s, b, d = hidden_states.shape
    x = hidden_states.reshape(s * b, d).astype(jnp.float32)
    return _route_hybrid(x, router_weight.astype(jnp.float32))
